# baseline (speedup 1.0000x reference)
import functools

import jax
import jax.numpy as jnp
from jax import lax
from jax.experimental import pallas as pl
from jax.experimental.pallas import tpu as pltpu
from jax.experimental.pallas import tpu_sc as plsc

Q = 1024
N = 1000
LANES = 16
QCHUNKS = Q // LANES
NC = 2
NS = 16
NW = NC * NS
J_PER_W = 32
NOUT = NW * LANES + 2 * LANES


def _sc_partials(rt, par):
    mesh = plsc.VectorSubcoreMesh(core_axis_name="c", subcore_axis_name="s")

    @functools.partial(
        pl.kernel,
        mesh=mesh,
        compiler_params=pltpu.CompilerParams(use_tc_tiling_on_sc=True,
                                             needs_layout_passes=False),
        out_type=jax.ShapeDtypeStruct((NOUT,), jnp.float32),
        scratch_types=[
            pltpu.VMEM((J_PER_W, Q), jnp.float32),
            pltpu.VMEM((J_PER_W, Q), jnp.float32),
            pltpu.VMEM((J_PER_W,), jnp.int32),
            pltpu.VMEM((7 * LANES,), jnp.int32),
            pltpu.VMEM((J_PER_W + LANES,), jnp.float32),
            pltpu.VMEM((LANES,), jnp.float32),
            pltpu.VMEM((2 * LANES,), jnp.float32),
            pltpu.SemaphoreType.DMA,
        ],
    )
    def k(rt_hbm, par_hbm, out_hbm, myrows_v, grows_v, par_v, par100_v,
          w_v, acc_v, cum_v, sem):
        wid = lax.axis_index("s") * NC + lax.axis_index("c")
        lo = jnp.minimum(wid * J_PER_W, N - J_PER_W)
        pltpu.sync_copy(par_hbm.at[pl.ds(lo, J_PER_W)], par_v)
        gather = pltpu.async_copy(rt_hbm.at[par_v], grows_v, sem)
        pltpu.sync_copy(rt_hbm.at[pl.ds(lo, J_PER_W)], myrows_v)

        lane = lax.iota(jnp.int32, LANES)
        for h in range(J_PER_W // LANES + 1):
            jg = lane + (lo + h * LANES)
            wvec = jnp.where((jg >= wid * J_PER_W) & (jg < lo + J_PER_W),
                             1.0 / (jg + 1).astype(jnp.float32), 0.0)
            w_v[pl.ds(h * LANES, LANES)] = wvec

        def j_body(m, acc):
            j = m * 2
            wpair = w_v[pl.ds(j, LANES)]
            t0 = jnp.zeros((LANES,), jnp.float32)
            t1 = jnp.zeros((LANES,), jnp.float32)
            for c in range(QCHUNKS):
                t0 = t0 + (myrows_v[j, pl.ds(c * LANES, LANES)]
                           * grows_v[j, pl.ds(c * LANES, LANES)])
                t1 = t1 + (myrows_v[j + 1, pl.ds(c * LANES, LANES)]
                           * grows_v[j + 1, pl.ds(c * LANES, LANES)])
            return acc + wpair[0] * t0 + wpair[1] * t1

        gather.wait()
        acc = lax.fori_loop(0, J_PER_W // 2, j_body,
                            jnp.zeros((LANES,), jnp.float32))
        acc_v[...] = acc
        pltpu.sync_copy(acc_v, out_hbm.at[pl.ds(wid * LANES, LANES)])

        @pl.when(wid == 0)
        def _():
            pltpu.sync_copy(par_hbm.at[pl.ds(0, 7 * LANES)], par100_v)
            cums = jnp.zeros((LANES,), jnp.float32)
            for c in range(6):
                cums = cums + par100_v[pl.ds(c * LANES, LANES)].astype(
                    jnp.float32)
            tail = par100_v[pl.ds(6 * LANES, LANES)].astype(jnp.float32)
            cums = cums + jnp.where(lane < 4, tail, 0.0)
            cum_v[pl.ds(0, LANES)] = cums
            cum_v[pl.ds(LANES, LANES)] = jnp.zeros((LANES,), jnp.float32)
            pltpu.sync_copy(cum_v, out_hbm.at[pl.ds(NW * LANES, 2 * LANES)])

    return k(rt, par)


def _tc_epilogue(partials):
    def body(p_ref, map_ref, cum_ref):
        total = jnp.sum(p_ref[pl.ds(0, NW * LANES)])
        cums = jnp.sum(p_ref[pl.ds(NW * LANES, 2 * LANES)])
        map_ref[...] = jnp.full((1,), total / (N * Q), jnp.float32)
        cum_ref[...] = jnp.full((Q,), cums / 100.0, jnp.float32)

    return pl.pallas_call(
        body,
        out_shape=(
            jax.ShapeDtypeStruct((1,), jnp.float32),
            jax.ShapeDtypeStruct((Q,), jnp.float32),
        ),
    )(partials)


def kernel(relevances, precision_at_ranks):
    rt = relevances.astype(jnp.float32).T
    par = precision_at_ranks.astype(jnp.int32)
    partials = _sc_partials(rt, par)
    map_out, cum_out = _tc_epilogue(partials)
    return (map_out[0], cum_out)


# speedup vs baseline: 1.1585x; 1.1585x over previous
import functools

import jax
import jax.numpy as jnp
from jax import lax
from jax.experimental import pallas as pl
from jax.experimental.pallas import tpu as pltpu
from jax.experimental.pallas import tpu_sc as plsc

Q = 1024
N = 1000
LANES = 16
QCHUNKS = Q // LANES
NC = 2
NS = 16
NW = NC * NS
J_PER_W = 32
NOUT = NW * LANES + 2 * LANES


def _sc_partials(rt, par):
    mesh = plsc.VectorSubcoreMesh(core_axis_name="c", subcore_axis_name="s")

    @functools.partial(
        pl.kernel,
        mesh=mesh,
        compiler_params=pltpu.CompilerParams(use_tc_tiling_on_sc=True,
                                             needs_layout_passes=False),
        out_type=jax.ShapeDtypeStruct((NOUT,), jnp.float32),
        scratch_types=[
            pltpu.VMEM((J_PER_W, Q), jnp.float32),
            pltpu.VMEM((J_PER_W, Q), jnp.float32),
            pltpu.VMEM((J_PER_W,), jnp.int32),
            pltpu.VMEM((7 * LANES,), jnp.int32),
            pltpu.VMEM((J_PER_W + LANES,), jnp.float32),
            pltpu.VMEM((LANES,), jnp.float32),
            pltpu.VMEM((2 * LANES,), jnp.float32),
            pltpu.SemaphoreType.DMA,
        ],
    )
    def k(rt_hbm, par_hbm, out_hbm, myrows_v, grows_v, par_v, par100_v,
          w_v, acc_v, cum_v, sem):
        wid = lax.axis_index("s") * NC + lax.axis_index("c")
        lo = jnp.minimum(wid * J_PER_W, N - J_PER_W)
        pltpu.sync_copy(par_hbm.at[pl.ds(lo, J_PER_W)], par_v)
        gather = pltpu.async_copy(rt_hbm.at[par_v], grows_v, sem)
        pltpu.sync_copy(rt_hbm.at[pl.ds(lo, J_PER_W)], myrows_v)

        lane = lax.iota(jnp.int32, LANES)
        for h in range(J_PER_W // LANES + 1):
            jg = lane + (lo + h * LANES)
            wvec = jnp.where((jg >= wid * J_PER_W) & (jg < lo + J_PER_W),
                             1.0 / (jg + 1).astype(jnp.float32), 0.0)
            w_v[pl.ds(h * LANES, LANES)] = wvec

        gather.wait()

        @plsc.parallel_loop(0, J_PER_W, carry=jnp.zeros((LANES,), jnp.float32))
        def acc(j, acc):
            wj = w_v[pl.ds(j, LANES)][0]
            t = jnp.zeros((LANES,), jnp.float32)
            for c in range(QCHUNKS):
                a = myrows_v[j, pl.ds(c * LANES, LANES)]
                b = grows_v[j, pl.ds(c * LANES, LANES)]
                t = t + a * b
            return acc + wj * t
        acc_v[...] = acc
        pltpu.sync_copy(acc_v, out_hbm.at[pl.ds(wid * LANES, LANES)])

        @pl.when(wid == 0)
        def _():
            pltpu.sync_copy(par_hbm.at[pl.ds(0, 7 * LANES)], par100_v)
            cums = jnp.zeros((LANES,), jnp.float32)
            for c in range(6):
                cums = cums + par100_v[pl.ds(c * LANES, LANES)].astype(
                    jnp.float32)
            tail = par100_v[pl.ds(6 * LANES, LANES)].astype(jnp.float32)
            cums = cums + jnp.where(lane < 4, tail, 0.0)
            cum_v[pl.ds(0, LANES)] = cums
            cum_v[pl.ds(LANES, LANES)] = jnp.zeros((LANES,), jnp.float32)
            pltpu.sync_copy(cum_v, out_hbm.at[pl.ds(NW * LANES, 2 * LANES)])

    return k(rt, par)


def _tc_epilogue(partials):
    def body(p_ref, map_ref, cum_ref):
        total = jnp.sum(p_ref[pl.ds(0, NW * LANES)])
        cums = jnp.sum(p_ref[pl.ds(NW * LANES, 2 * LANES)])
        map_ref[...] = jnp.full((1,), total / (N * Q), jnp.float32)
        cum_ref[...] = jnp.full((Q,), cums / 100.0, jnp.float32)

    return pl.pallas_call(
        body,
        out_shape=(
            jax.ShapeDtypeStruct((1,), jnp.float32),
            jax.ShapeDtypeStruct((Q,), jnp.float32),
        ),
    )(partials)


def kernel(relevances, precision_at_ranks):
    rt = relevances.astype(jnp.float32).T
    par = precision_at_ranks.astype(jnp.int32)
    partials = _sc_partials(rt, par)
    map_out, cum_out = _tc_epilogue(partials)
    return (map_out[0], cum_out)
